# baseline (speedup 1.0000x reference)
import functools

import jax
import jax.numpy as jnp
from jax import lax
from jax.experimental import pallas as pl
from jax.experimental.pallas import tpu as pltpu
from jax.experimental.pallas import tpu_sc as plsc

N_NEG = 50
D = 32
L = 16
WPT = 4
CTX_SLOT = 56

_LOG1P = (
    0.9999992249459306,
    -0.4999677773260973,
    0.3328626878693824,
    -0.2465484102008357,
    0.18517671305376252,
    -0.12601773504363445,
    0.0671992182399122,
    -0.023381649402895242,
    0.003824912525210834,
)


def _softplus(x):
    u = jnp.exp(-jnp.abs(x))
    p = jnp.full((L,), _LOG1P[-1], jnp.float32)
    for c in _LOG1P[-2::-1]:
        p = p * u + c
    return jnp.maximum(x, 0.0) + p * u


def _fold(v, z):
    for s in (8, 4, 2, 1):
        z[pl.ds(0, L)] = v
        v = v + z[pl.ds(s, L)]
    return v


def _aligned(idx):
    return pl.multiple_of(idx & -128, 128)


def _pick(vec, sub, ebuf):
    ebuf[pl.ds(0, L)] = vec
    return ebuf[pl.ds(sub, L)][0]


def _sc_body(tgt_hbm, ctx_hbm, tw_hbm, cw_hbm, neg_hbm, out_hbm,
             tw_v, widx, tblk, wblks, ebuf, z, pstage, pbuf, out_v, spm, sem):
    core = lax.axis_index("c")
    sid = lax.axis_index("s")

    @pl.when(core == 0)
    def _():
        di0 = pltpu.async_copy(tw_hbm, tw_v.at[pl.ds(0, 1)], sem)
        di1 = pltpu.async_copy(neg_hbm, widx.at[pl.ds(0, N_NEG)], sem)
        di2 = pltpu.async_copy(cw_hbm, widx.at[pl.ds(CTX_SLOT, 1)], sem)
        z[pl.ds(L, L)] = jnp.zeros((L,), jnp.float32)
        di0.wait()

        it0 = tw_v[pl.ds(0, L)][0]
        copies = [pltpu.async_copy(
            tgt_hbm.at[:, pl.ds(_aligned(it0), 128)], tblk, sem)]

        di1.wait()
        di2.wait()
        wv = widx[pl.ds(sid * WPT, L)]
        slot0 = sid * WPT
        raw = [wv[j] for j in range(WPT)]
        gids = [slot0 + j for j in range(WPT)]
        valid = [(g < N_NEG) | (g == CTX_SLOT) for g in gids]
        idxs = [jnp.where(valid[j], raw[j], 0) for j in range(WPT)]

        for j in range(WPT):
            copies.append(pltpu.async_copy(
                ctx_hbm.at[:, pl.ds(_aligned(idxs[j]), 128)],
                wblks.at[j], sem))
        for c in copies:
            c.wait()

        t_off = it0 & 112
        t_sub = it0 & 15
        t_sc = [_pick(tblk[d, pl.ds(t_off, L)], t_sub, ebuf) for d in range(D)]

        lane = lax.iota(jnp.int32, L)
        xv = jnp.full((L,), -30.0, jnp.float32)
        for j in range(WPT):
            off = idxs[j] & 112
            sub = idxs[j] & 15
            acc = jnp.zeros((L,), jnp.float32)
            for d in range(D):
                acc = acc + t_sc[d] * wblks[j, d, pl.ds(off, L)]
            dot = _pick(acc, sub, ebuf)
            g = gids[j]
            x = jnp.where(g == CTX_SLOT, -dot,
                          jnp.where(g < N_NEG, dot, jnp.float32(-30.0)))
            xv = jnp.where(lane == j, x, xv)

        pstage[...] = _softplus(xv)
        pltpu.sync_copy(pstage, spm.at[sid * L + lane])
        plsc.subcore_barrier()

        @pl.when(sid == 0)
        def _():
            pltpu.sync_copy(spm, pbuf)
            total = pbuf[pl.ds(0, L)]
            for t in range(1, L):
                total = total + pbuf[pl.ds(t * L, L)]
            out_v[...] = _fold(total, z)
            pltpu.sync_copy(out_v, out_hbm)


@functools.cache
def _build():
    mesh = plsc.VectorSubcoreMesh(core_axis_name="c", subcore_axis_name="s",
                                  num_cores=1)
    return pl.kernel(
        _sc_body,
        out_type=jax.ShapeDtypeStruct((L,), jnp.float32),
        mesh=mesh,
        scratch_types=[
            pltpu.VMEM((L,), jnp.int32),
            pltpu.VMEM((5 * L,), jnp.int32),
            pltpu.VMEM((D, 128), jnp.float32),
            pltpu.VMEM((WPT, D, 128), jnp.float32),
            pltpu.VMEM((2 * L,), jnp.float32),
            pltpu.VMEM((2 * L,), jnp.float32),
            pltpu.VMEM((L,), jnp.float32),
            pltpu.VMEM((L * L,), jnp.float32),
            pltpu.VMEM((L,), jnp.float32),
            pltpu.VMEM_SHARED((L * L,), jnp.float32),
            pltpu.SemaphoreType.DMA,
        ],
        compiler_params=pltpu.CompilerParams(
            disable_bounds_checks=True,
            disable_semaphore_checks=True,
            skip_device_barrier=True,
        ),
    )


def kernel(embeddings_target, embeddings_context, target_word, context_word,
           negative_context_words):
    out = _build()(embeddings_target.T, embeddings_context.T, target_word,
                   context_word, negative_context_words)
    return out[0]
